# baseline (speedup 1.0000x reference)
import functools

import jax
import jax.numpy as jnp
from jax import lax
from jax.experimental import pallas as pl
from jax.experimental.pallas import tpu as pltpu
from jax.experimental.pallas import tpu_sc as plsc


_RB = 4096


def _project_pack(tT, Wt, b2):
    D, V = tT.shape
    O = Wt.shape[1]
    H = O // 2
    R = _RB
    NB = pl.cdiv(V, R)

    def body(x_ref, wt_ref, b_ref, o_ref):
        z = (
            lax.dot_general(
                x_ref[...], wt_ref[...], (((0,), (0,)), ((), ())),
                preferred_element_type=jnp.float32,
            )
            + b_ref[...]
        )
        u = lax.bitcast_convert_type(z, jnp.uint32)
        rn = (u + 0x7FFF + ((u >> 16) & 1)) >> 16
        w = lax.bitcast_convert_type(rn[:, :H] | (rn[:, H:] << 16), jnp.int32)
        o_ref[...] = jnp.concatenate([w[: R // 2], w[R // 2 :]], axis=1)

    return pl.pallas_call(
        body,
        grid=(NB,),
        in_specs=[
            pl.BlockSpec((D, R), lambda i: (0, i)),
            pl.BlockSpec((D, O), lambda i: (0, 0)),
            pl.BlockSpec((1, O), lambda i: (0, 0)),
        ],
        out_specs=pl.BlockSpec((R // 2, 2 * H), lambda i: (i, 0)),
        out_shape=jax.ShapeDtypeStruct((NB * (R // 2), 2 * H), jnp.int32),
    )(tT, Wt, b2)


def _remap_idx(idx):
    R = _RB
    return (idx & ~(R - 1)) | ((idx & (R // 2 - 1)) << 1) | ((idx >> 11) & 1)


@functools.lru_cache(maxsize=None)
def _make_sc_gather_unpack(V, O, N):
    H = O // 2
    info = plsc.get_sparse_core_info()
    NC, NS = info.num_cores, info.num_subcores
    NW = NC * NS
    n_per_w = N // NW
    CH = 256
    IR = CH // 128
    n_chunks = n_per_w // CH
    ir_per_w = n_per_w // 128
    QH = H // 16
    mesh = plsc.VectorSubcoreMesh(core_axis_name="c", subcore_axis_name="s")

    @functools.partial(
        pl.kernel,
        mesh=mesh,
        out_type=jax.ShapeDtypeStruct((N, O), jnp.float32),
        scratch_types=[
            pltpu.VMEM((2, IR, 128), jnp.int32),
            pltpu.VMEM((2, CH, H), jnp.int32),
            pltpu.VMEM((CH, O), jnp.float32),
            pltpu.SemaphoreType.DMA,
            pltpu.SemaphoreType.DMA,
        ],
        compiler_params=pltpu.CompilerParams(
            use_tc_tiling_on_sc=False, needs_layout_passes=False
        ),
    )
    def gather(packed_hbm, idx_hbm, out_hbm, idx_v, pk_v, rows_v, gsem, ssem):
        wid = lax.axis_index("s") * NC + lax.axis_index("c")
        mask_hi = jnp.full((16,), -65536, jnp.int32)
        n_pairs = n_chunks // 2

        def start_chunk(k, buf):
            irow = wid * ir_per_w + k * IR
            pltpu.sync_copy(idx_hbm.at[pl.ds(irow, IR)], idx_v.at[buf])
            for j in range(IR):
                pltpu.async_copy(
                    packed_hbm.at[idx_v.at[buf, j]],
                    pk_v.at[buf, pl.ds(j * 128, 128)],
                    gsem,
                )

        def drain_unpack_store(k, buf):
            for j in range(IR):
                pltpu.make_async_copy(
                    packed_hbm.at[pl.ds(0, 128)],
                    pk_v.at[buf, pl.ds(j * 128, 128)],
                    gsem,
                ).wait()

            def row(r, carry):
                for q in range(QH):
                    w = pk_v[buf, r, pl.ds(q * 16, 16)]
                    rows_v[r, pl.ds(q * 16, 16)] = plsc.bitcast(
                        w << 16, jnp.float32
                    )
                    rows_v[r, pl.ds(H + q * 16, 16)] = plsc.bitcast(
                        w & mask_hi, jnp.float32
                    )
                return carry

            lax.fori_loop(0, CH, row, 0)
            rbase = wid * n_per_w + k * CH
            pltpu.async_copy(rows_v, out_hbm.at[pl.ds(rbase, CH)], ssem).wait()

        def pair(kk, carry):
            k0 = kk * 2
            start_chunk(k0 + 1, 1)
            drain_unpack_store(k0, 0)

            @pl.when(kk < n_pairs - 1)
            def _():
                start_chunk(k0 + 2, 0)

            drain_unpack_store(k0 + 1, 1)
            return carry

        start_chunk(0, 0)
        lax.fori_loop(0, n_pairs, pair, 0)

    return gather


def kernel(t_id, table, W, b):
    B, L = t_id.shape
    V, D = table.shape
    O = W.shape[0]
    N = B * L
    packed = _project_pack(table.T, W.T, b.reshape(1, O))
    pview = packed.reshape(packed.shape[0] * 2, O // 2)
    idx = _remap_idx(t_id.T.astype(jnp.int32)).reshape(N // 128, 128)
    outT = _make_sc_gather_unpack(pview.shape[0], O, N)(pview, idx)
    return outT.reshape(L, B, O).transpose(1, 0, 2)


# speedup vs baseline: 1.1452x; 1.1452x over previous
import functools

import jax
import jax.numpy as jnp
from jax import lax
from jax.experimental import pallas as pl
from jax.experimental.pallas import tpu as pltpu
from jax.experimental.pallas import tpu_sc as plsc


_RB = 4096


def _project_pack(tT, Wt, b2):
    D, V = tT.shape
    O = Wt.shape[1]
    H = O // 2
    R = _RB
    NB = pl.cdiv(V, R)

    def body(x_ref, wt_ref, b_ref, o_ref):
        z = (
            lax.dot_general(
                x_ref[...].astype(jnp.bfloat16),
                wt_ref[...].astype(jnp.bfloat16),
                (((0,), (0,)), ((), ())),
                preferred_element_type=jnp.float32,
            )
            + b_ref[...]
        )
        u = lax.bitcast_convert_type(z, jnp.uint32)
        rn = (u + 0x7FFF + ((u >> 16) & 1)) >> 16
        w = lax.bitcast_convert_type(rn[:, :H] | (rn[:, H:] << 16), jnp.int32)
        o_ref[...] = jnp.concatenate([w[: R // 2], w[R // 2 :]], axis=1)

    return pl.pallas_call(
        body,
        grid=(NB,),
        in_specs=[
            pl.BlockSpec((D, R), lambda i: (0, i)),
            pl.BlockSpec((D, O), lambda i: (0, 0)),
            pl.BlockSpec((1, O), lambda i: (0, 0)),
        ],
        out_specs=pl.BlockSpec((R // 2, 2 * H), lambda i: (i, 0)),
        out_shape=jax.ShapeDtypeStruct((NB * (R // 2), 2 * H), jnp.int32),
    )(tT, Wt, b2)


def _remap_idx(idx):
    R = _RB
    return (idx & ~(R - 1)) | ((idx & (R // 2 - 1)) << 1) | ((idx >> 11) & 1)


@functools.lru_cache(maxsize=None)
def _make_sc_gather_unpack(V, O, N):
    H = O // 2
    info = plsc.get_sparse_core_info()
    NC, NS = info.num_cores, info.num_subcores
    NW = NC * NS
    n_per_w = N // NW
    CH = 256
    IR = CH // 128
    n_chunks = n_per_w // CH
    ir_per_w = n_per_w // 128
    QH = H // 16
    mesh = plsc.VectorSubcoreMesh(core_axis_name="c", subcore_axis_name="s")

    @functools.partial(
        pl.kernel,
        mesh=mesh,
        out_type=jax.ShapeDtypeStruct((N, O), jnp.float32),
        scratch_types=[
            pltpu.VMEM((2, IR, 128), jnp.int32),
            pltpu.VMEM((2, CH, H), jnp.int32),
            pltpu.VMEM((2, CH, O), jnp.float32),
            pltpu.SemaphoreType.DMA,
            pltpu.SemaphoreType.DMA,
        ],
        compiler_params=pltpu.CompilerParams(
            use_tc_tiling_on_sc=False, needs_layout_passes=False
        ),
    )
    def gather(packed_hbm, idx_hbm, out_hbm, idx_v, pk_v, rows_v, gsem, ssem):
        wid = lax.axis_index("s") * NC + lax.axis_index("c")
        mask_hi = jnp.full((16,), -65536, jnp.int32)
        n_pairs = n_chunks // 2

        def start_chunk(k, buf):
            irow = wid * ir_per_w + k * IR
            pltpu.sync_copy(idx_hbm.at[pl.ds(irow, IR)], idx_v.at[buf])
            for j in range(IR):
                pltpu.async_copy(
                    packed_hbm.at[idx_v.at[buf, j]],
                    pk_v.at[buf, pl.ds(j * 128, 128)],
                    gsem,
                )

        def drain_store(buf):
            pltpu.make_async_copy(
                out_hbm.at[pl.ds(0, CH)], rows_v.at[buf], ssem
            ).wait()

        def drain_unpack_store(k, buf):
            for j in range(IR):
                pltpu.make_async_copy(
                    packed_hbm.at[pl.ds(0, 128)],
                    pk_v.at[buf, pl.ds(j * 128, 128)],
                    gsem,
                ).wait()

            def rows4(r4, carry):
                for dr in range(4):
                    r = r4 * 4 + dr
                    for q in range(QH):
                        w = pk_v[buf, r, pl.ds(q * 16, 16)]
                        rows_v[buf, r, pl.ds(q * 16, 16)] = plsc.bitcast(
                            w << 16, jnp.float32
                        )
                        rows_v[buf, r, pl.ds(H + q * 16, 16)] = plsc.bitcast(
                            w & mask_hi, jnp.float32
                        )
                return carry

            lax.fori_loop(0, CH // 4, rows4, 0)
            rbase = wid * n_per_w + k * CH
            pltpu.async_copy(rows_v.at[buf], out_hbm.at[pl.ds(rbase, CH)], ssem)

        def pair(kk, carry):
            k0 = kk * 2
            start_chunk(k0 + 1, 1)

            @pl.when(kk > 0)
            def _():
                drain_store(0)

            drain_unpack_store(k0, 0)

            @pl.when(kk < n_pairs - 1)
            def _():
                start_chunk(k0 + 2, 0)

            @pl.when(kk > 0)
            def _():
                drain_store(1)

            drain_unpack_store(k0 + 1, 1)
            return carry

        start_chunk(0, 0)
        lax.fori_loop(0, n_pairs, pair, 0)
        drain_store(0)
        drain_store(1)

    return gather


def kernel(t_id, table, W, b):
    B, L = t_id.shape
    V, D = table.shape
    O = W.shape[0]
    N = B * L
    packed = _project_pack(table.T, W.T, b.reshape(1, O))
    pview = packed.reshape(packed.shape[0] * 2, O // 2)
    idx = _remap_idx(t_id.T.astype(jnp.int32)).reshape(N // 128, 128)
    outT = _make_sc_gather_unpack(pview.shape[0], O, N)(pview, idx)
    return outT.reshape(L, B, O).transpose(1, 0, 2)


# speedup vs baseline: 1.2079x; 1.0547x over previous
import functools

import jax
import jax.numpy as jnp
from jax import lax
from jax.experimental import pallas as pl
from jax.experimental.pallas import tpu as pltpu
from jax.experimental.pallas import tpu_sc as plsc


_RB = 4096


def _project_pack(tT, Wt, b2):
    D, V = tT.shape
    O = Wt.shape[1]
    H = O // 2
    R = _RB
    NB = pl.cdiv(V, R)

    def body(x_ref, wt_ref, b_ref, o_ref):
        xb = x_ref[...].astype(jnp.bfloat16)
        wb = wt_ref[...].astype(jnp.bfloat16)

        def pack_half(xh):
            z = (
                lax.dot_general(
                    xh, wb, (((0,), (0,)), ((), ())),
                    preferred_element_type=jnp.float32,
                )
                + b_ref[...]
            )
            u = lax.bitcast_convert_type(z, jnp.uint32)
            rn = (u + 0x7FFF + ((u >> 16) & 1)) >> 16
            return lax.bitcast_convert_type(
                rn[:, :H] | (rn[:, H:] << 16), jnp.int32
            )

        o_ref[:, :H] = pack_half(xb[:, : R // 2])
        o_ref[:, H:] = pack_half(xb[:, R // 2 :])

    return pl.pallas_call(
        body,
        grid=(NB,),
        in_specs=[
            pl.BlockSpec((D, R), lambda i: (0, i)),
            pl.BlockSpec((D, O), lambda i: (0, 0)),
            pl.BlockSpec((1, O), lambda i: (0, 0)),
        ],
        out_specs=pl.BlockSpec((R // 2, 2 * H), lambda i: (i, 0)),
        out_shape=jax.ShapeDtypeStruct((NB * (R // 2), 2 * H), jnp.int32),
    )(tT, Wt, b2)


def _remap_idx(idx):
    R = _RB
    return (idx & ~(R - 1)) | ((idx & (R // 2 - 1)) << 1) | ((idx >> 11) & 1)


@functools.lru_cache(maxsize=None)
def _make_sc_gather_unpack(V, O, N):
    H = O // 2
    info = plsc.get_sparse_core_info()
    NC, NS = info.num_cores, info.num_subcores
    NW = NC * NS
    n_per_w = N // NW
    CH = 256
    IR = CH // 128
    n_chunks = n_per_w // CH
    ir_per_w = n_per_w // 128
    QH = H // 16
    mesh = plsc.VectorSubcoreMesh(core_axis_name="c", subcore_axis_name="s")

    @functools.partial(
        pl.kernel,
        mesh=mesh,
        out_type=jax.ShapeDtypeStruct((N, O), jnp.float32),
        scratch_types=[
            pltpu.VMEM((ir_per_w, 128), jnp.int32),
            pltpu.VMEM((2, CH, H), jnp.int32),
            pltpu.VMEM((2, CH, O), jnp.float32),
            pltpu.SemaphoreType.DMA,
            pltpu.SemaphoreType.DMA,
        ],
        compiler_params=pltpu.CompilerParams(
            use_tc_tiling_on_sc=False, needs_layout_passes=False
        ),
    )
    def gather(packed_hbm, idx_hbm, out_hbm, idx_v, pk_v, rows_v, gsem, ssem):
        wid = lax.axis_index("s") * NC + lax.axis_index("c")
        mask_hi = jnp.full((16,), -65536, jnp.int32)
        n_pairs = n_chunks // 2

        def start_chunk(k, buf):
            for j in range(IR):
                pltpu.async_copy(
                    packed_hbm.at[idx_v.at[k * IR + j]],
                    pk_v.at[buf, pl.ds(j * 128, 128)],
                    gsem,
                )

        def drain_store(buf):
            pltpu.make_async_copy(
                out_hbm.at[pl.ds(0, CH)], rows_v.at[buf], ssem
            ).wait()

        def drain_unpack_store(k, buf):
            for j in range(IR):
                pltpu.make_async_copy(
                    packed_hbm.at[pl.ds(0, 128)],
                    pk_v.at[buf, pl.ds(j * 128, 128)],
                    gsem,
                ).wait()

            def rows4(r4, carry):
                for dr in range(4):
                    r = r4 * 4 + dr
                    for q in range(QH):
                        w = pk_v[buf, r, pl.ds(q * 16, 16)]
                        rows_v[buf, r, pl.ds(q * 16, 16)] = plsc.bitcast(
                            w << 16, jnp.float32
                        )
                        rows_v[buf, r, pl.ds(H + q * 16, 16)] = plsc.bitcast(
                            w & mask_hi, jnp.float32
                        )
                return carry

            lax.fori_loop(0, CH // 4, rows4, 0)
            rbase = wid * n_per_w + k * CH
            pltpu.async_copy(rows_v.at[buf], out_hbm.at[pl.ds(rbase, CH)], ssem)

        def pair(kk, carry):
            k0 = kk * 2
            start_chunk(k0 + 1, 1)

            @pl.when(kk > 0)
            def _():
                drain_store(0)

            drain_unpack_store(k0, 0)

            @pl.when(kk < n_pairs - 1)
            def _():
                start_chunk(k0 + 2, 0)

            @pl.when(kk > 0)
            def _():
                drain_store(1)

            drain_unpack_store(k0 + 1, 1)
            return carry

        pltpu.sync_copy(idx_hbm.at[pl.ds(wid * ir_per_w, ir_per_w)], idx_v)
        start_chunk(0, 0)
        lax.fori_loop(0, n_pairs, pair, 0)
        drain_store(0)
        drain_store(1)

    return gather


def kernel(t_id, table, W, b):
    B, L = t_id.shape
    V, D = table.shape
    O = W.shape[0]
    N = B * L
    packed = _project_pack(table.T, W.T, b.reshape(1, O))
    pview = packed.reshape(packed.shape[0] * 2, O // 2)
    idx = _remap_idx(t_id.T.astype(jnp.int32)).reshape(N // 128, 128)
    outT = _make_sc_gather_unpack(pview.shape[0], O, N)(pview, idx)
    return outT.reshape(L, B, O).transpose(1, 0, 2)


# speedup vs baseline: 1.3568x; 1.1233x over previous
import functools

import jax
import jax.numpy as jnp
from jax import lax
from jax.experimental import pallas as pl
from jax.experimental.pallas import tpu as pltpu
from jax.experimental.pallas import tpu_sc as plsc


_RB = 8192


def _project_pack(tT, Wt, b2):
    D, V = tT.shape
    O = Wt.shape[1]
    H = O // 2
    R = _RB
    NB = pl.cdiv(V, R)

    def body(x_ref, wt_ref, b_ref, o_ref):
        xb = x_ref[...].astype(jnp.bfloat16)
        wb = wt_ref[...].astype(jnp.bfloat16)

        def pack_half(xh):
            z = (
                lax.dot_general(
                    xh, wb, (((0,), (0,)), ((), ())),
                    preferred_element_type=jnp.float32,
                )
                + b_ref[...]
            )
            u = lax.bitcast_convert_type(z, jnp.uint32)
            return lax.bitcast_convert_type(
                (u[:, :H] >> 16) | (u[:, H:] & jnp.uint32(0xFFFF0000)),
                jnp.int32,
            )

        o_ref[:, :H] = pack_half(xb[:, : R // 2])
        o_ref[:, H:] = pack_half(xb[:, R // 2 :])

    return pl.pallas_call(
        body,
        grid=(NB,),
        in_specs=[
            pl.BlockSpec((D, R), lambda i: (0, i)),
            pl.BlockSpec((D, O), lambda i: (0, 0)),
            pl.BlockSpec((1, O), lambda i: (0, 0)),
        ],
        out_specs=pl.BlockSpec((R // 2, 2 * H), lambda i: (i, 0)),
        out_shape=jax.ShapeDtypeStruct((NB * (R // 2), 2 * H), jnp.int32),
    )(tT, Wt, b2)


def _remap_idx(idx):
    R = _RB
    sh = (R // 2).bit_length() - 1
    return (idx & ~(R - 1)) | ((idx & (R // 2 - 1)) << 1) | ((idx >> sh) & 1)


@functools.lru_cache(maxsize=None)
def _make_sc_gather_unpack(V, O, N):
    H = O // 2
    info = plsc.get_sparse_core_info()
    NC, NS = info.num_cores, info.num_subcores
    NW = NC * NS
    n_per_w = N // NW
    CH = 256
    IR = CH // 128
    n_chunks = n_per_w // CH
    ir_per_w = n_per_w // 128
    QH = H // 16
    mesh = plsc.VectorSubcoreMesh(core_axis_name="c", subcore_axis_name="s")

    @functools.partial(
        pl.kernel,
        mesh=mesh,
        out_type=jax.ShapeDtypeStruct((N, O), jnp.float32),
        scratch_types=[
            pltpu.VMEM((ir_per_w, 128), jnp.int32),
            pltpu.VMEM((2, CH, H), jnp.int32),
            pltpu.VMEM((2, CH, O), jnp.float32),
            pltpu.SemaphoreType.DMA,
            pltpu.SemaphoreType.DMA,
        ],
        compiler_params=pltpu.CompilerParams(
            use_tc_tiling_on_sc=False, needs_layout_passes=False
        ),
    )
    def gather(packed_hbm, idx_hbm, out_hbm, idx_v, pk_v, rows_v, gsem, ssem):
        wid = lax.axis_index("s") * NC + lax.axis_index("c")
        mask_hi = jnp.full((16,), -65536, jnp.int32)
        n_pairs = n_chunks // 2

        def start_chunk(k, buf):
            for j in range(IR):
                pltpu.async_copy(
                    packed_hbm.at[idx_v.at[k * IR + j]],
                    pk_v.at[buf, pl.ds(j * 128, 128)],
                    gsem,
                )

        def drain_store(buf):
            pltpu.make_async_copy(
                out_hbm.at[pl.ds(0, CH)], rows_v.at[buf], ssem
            ).wait()

        def drain_unpack_store(k, buf):
            for j in range(IR):
                pltpu.make_async_copy(
                    packed_hbm.at[pl.ds(0, 128)],
                    pk_v.at[buf, pl.ds(j * 128, 128)],
                    gsem,
                ).wait()

            def rows4(r4, carry):
                for dr in range(4):
                    r = r4 * 4 + dr
                    for q in range(QH):
                        w = pk_v[buf, r, pl.ds(q * 16, 16)]
                        rows_v[buf, r, pl.ds(q * 16, 16)] = plsc.bitcast(
                            w << 16, jnp.float32
                        )
                        rows_v[buf, r, pl.ds(H + q * 16, 16)] = plsc.bitcast(
                            w & mask_hi, jnp.float32
                        )
                return carry

            lax.fori_loop(0, CH // 4, rows4, 0)
            rbase = wid * n_per_w + k * CH
            pltpu.async_copy(rows_v.at[buf], out_hbm.at[pl.ds(rbase, CH)], ssem)

        def pair(kk, carry):
            k0 = kk * 2
            start_chunk(k0 + 1, 1)

            @pl.when(kk > 0)
            def _():
                drain_store(0)

            drain_unpack_store(k0, 0)

            @pl.when(kk < n_pairs - 1)
            def _():
                start_chunk(k0 + 2, 0)

            @pl.when(kk > 0)
            def _():
                drain_store(1)

            drain_unpack_store(k0 + 1, 1)
            return carry

        pltpu.sync_copy(idx_hbm.at[pl.ds(wid * ir_per_w, ir_per_w)], idx_v)
        start_chunk(0, 0)
        lax.fori_loop(0, n_pairs, pair, 0)
        drain_store(0)
        drain_store(1)

    return gather


def kernel(t_id, table, W, b):
    B, L = t_id.shape
    V, D = table.shape
    O = W.shape[0]
    N = B * L
    packed = _project_pack(table.T, W.T, b.reshape(1, O))
    pview = packed.reshape(packed.shape[0] * 2, O // 2)
    idx = _remap_idx(t_id.T.astype(jnp.int32)).reshape(N // 128, 128)
    outT = _make_sc_gather_unpack(pview.shape[0], O, N)(pview, idx)
    return outT.reshape(L, B, O).transpose(1, 0, 2)


# speedup vs baseline: 1.9727x; 1.4540x over previous
import functools

import jax
import jax.numpy as jnp
from jax import lax
from jax.experimental import pallas as pl
from jax.experimental.pallas import tpu as pltpu
from jax.experimental.pallas import tpu_sc as plsc


_RB = 8192


def _project_pack(tT, Wt, b2):
    D, V = tT.shape
    O = Wt.shape[1]
    H = O // 2
    R = _RB
    NB = pl.cdiv(V, R)

    def body(x_ref, wt_ref, b_ref, o_ref):
        xb = x_ref[...].astype(jnp.bfloat16)
        wb = wt_ref[...].astype(jnp.bfloat16)

        def pack_half(xh):
            z = (
                lax.dot_general(
                    xh, wb, (((0,), (0,)), ((), ())),
                    preferred_element_type=jnp.float32,
                )
                + b_ref[...]
            )
            u = lax.bitcast_convert_type(z, jnp.uint32)
            return lax.bitcast_convert_type(
                (u[:, :H] >> 16) | (u[:, H:] & jnp.uint32(0xFFFF0000)),
                jnp.int32,
            )

        o_ref[:, :H] = pack_half(xb[:, : R // 2])
        o_ref[:, H:] = pack_half(xb[:, R // 2 :])

    return pl.pallas_call(
        body,
        grid=(NB,),
        in_specs=[
            pl.BlockSpec((D, R), lambda i: (0, i)),
            pl.BlockSpec((D, O), lambda i: (0, 0)),
            pl.BlockSpec((1, O), lambda i: (0, 0)),
        ],
        out_specs=pl.BlockSpec((R // 2, 2 * H), lambda i: (i, 0)),
        out_shape=jax.ShapeDtypeStruct((NB * (R // 2), 2 * H), jnp.int32),
    )(tT, Wt, b2)


def _remap_idx(idx):
    R = _RB
    sh = (R // 2).bit_length() - 1
    return (idx & ~(R - 1)) | ((idx & (R // 2 - 1)) << 1) | ((idx >> sh) & 1)


@functools.lru_cache(maxsize=None)
def _make_sc_gather_unpack(V, O, N):
    H = O // 2
    info = plsc.get_sparse_core_info()
    NC, NS = info.num_cores, info.num_subcores
    NW = NC * NS
    n_per_w = N // NW
    CH = 256
    IR = CH // 128
    n_chunks = n_per_w // CH
    ir_per_w = n_per_w // 128
    QH = H // 16
    mesh = plsc.VectorSubcoreMesh(core_axis_name="c", subcore_axis_name="s")

    @functools.partial(
        pl.kernel,
        mesh=mesh,
        out_type=jax.ShapeDtypeStruct((N, O), jnp.float32),
        scratch_types=[
            pltpu.VMEM((ir_per_w, 128), jnp.int32),
            pltpu.VMEM((2, CH, H), jnp.int32),
            pltpu.VMEM((2, CH, O), jnp.float32),
            pltpu.SemaphoreType.DMA,
            pltpu.SemaphoreType.DMA,
        ],
        compiler_params=pltpu.CompilerParams(
            use_tc_tiling_on_sc=False, needs_layout_passes=False
        ),
    )
    def gather(packed_hbm, idx_hbm, out_hbm, idx_v, pk_v, rows_v, gsem, ssem):
        wid = lax.axis_index("s") * NC + lax.axis_index("c")
        mask_hi = jnp.full((16,), -65536, jnp.int32)
        n_pairs = n_chunks // 2

        def start_chunk(k, buf):
            for j in range(IR):
                pltpu.async_copy(
                    packed_hbm.at[idx_v.at[k * IR + j]],
                    pk_v.at[buf, pl.ds(j * 128, 128)],
                    gsem,
                )

        def drain_store(buf):
            pltpu.make_async_copy(
                out_hbm.at[pl.ds(0, CH)], rows_v.at[buf], ssem
            ).wait()

        def drain_unpack_store(k, buf):
            for j in range(IR):
                pltpu.make_async_copy(
                    packed_hbm.at[pl.ds(0, 128)],
                    pk_v.at[buf, pl.ds(j * 128, 128)],
                    gsem,
                ).wait()

            @plsc.parallel_loop(0, CH, 1, unroll=8)
            def _(r):
                for q in range(QH):
                    w = pk_v[buf, r, pl.ds(q * 16, 16)]
                    rows_v[buf, r, pl.ds(q * 16, 16)] = plsc.bitcast(
                        w << 16, jnp.float32
                    )
                    rows_v[buf, r, pl.ds(H + q * 16, 16)] = plsc.bitcast(
                        w & mask_hi, jnp.float32
                    )
            rbase = wid * n_per_w + k * CH
            pltpu.async_copy(rows_v.at[buf], out_hbm.at[pl.ds(rbase, CH)], ssem)

        def pair(kk, carry):
            k0 = kk * 2
            start_chunk(k0 + 1, 1)

            @pl.when(kk > 0)
            def _():
                drain_store(0)

            drain_unpack_store(k0, 0)

            @pl.when(kk < n_pairs - 1)
            def _():
                start_chunk(k0 + 2, 0)

            @pl.when(kk > 0)
            def _():
                drain_store(1)

            drain_unpack_store(k0 + 1, 1)
            return carry

        pltpu.sync_copy(idx_hbm.at[pl.ds(wid * ir_per_w, ir_per_w)], idx_v)
        start_chunk(0, 0)
        lax.fori_loop(0, n_pairs, pair, 0)
        drain_store(0)
        drain_store(1)

    return gather


def kernel(t_id, table, W, b):
    B, L = t_id.shape
    V, D = table.shape
    O = W.shape[0]
    N = B * L
    packed = _project_pack(table.T, W.T, b.reshape(1, O))
    pview = packed.reshape(packed.shape[0] * 2, O // 2)
    idx = _remap_idx(t_id.T.astype(jnp.int32)).reshape(N // 128, 128)
    outT = _make_sc_gather_unpack(pview.shape[0], O, N)(pview, idx)
    return outT.reshape(L, B, O).transpose(1, 0, 2)


# speedup vs baseline: 2.0926x; 1.0607x over previous
import functools

import jax
import jax.numpy as jnp
from jax import lax
from jax.experimental import pallas as pl
from jax.experimental.pallas import tpu as pltpu
from jax.experimental.pallas import tpu_sc as plsc


_RB = 16384


def _project_pack(tT, Wt, b2):
    D, V = tT.shape
    O = Wt.shape[1]
    H = O // 2
    R = _RB
    NB = pl.cdiv(V, R)

    def body(x_ref, wt_ref, b_ref, o_ref):
        xb = x_ref[...].astype(jnp.bfloat16)
        wb = wt_ref[...].astype(jnp.bfloat16)

        def pack_half(xh):
            z = (
                lax.dot_general(
                    xh, wb, (((0,), (0,)), ((), ())),
                    preferred_element_type=jnp.float32,
                )
                + b_ref[...]
            )
            u = lax.bitcast_convert_type(z, jnp.uint32)
            return lax.bitcast_convert_type(
                (u[:, :H] >> 16) | (u[:, H:] & jnp.uint32(0xFFFF0000)),
                jnp.int32,
            )

        o_ref[:, :H] = pack_half(xb[:, : R // 2])
        o_ref[:, H:] = pack_half(xb[:, R // 2 :])

    return pl.pallas_call(
        body,
        grid=(NB,),
        in_specs=[
            pl.BlockSpec((D, R), lambda i: (0, i)),
            pl.BlockSpec((D, O), lambda i: (0, 0)),
            pl.BlockSpec((1, O), lambda i: (0, 0)),
        ],
        out_specs=pl.BlockSpec((R // 2, 2 * H), lambda i: (i, 0)),
        out_shape=jax.ShapeDtypeStruct((NB * (R // 2), 2 * H), jnp.int32),
    )(tT, Wt, b2)


def _remap_idx(idx):
    R = _RB
    sh = (R // 2).bit_length() - 1
    return (idx & ~(R - 1)) | ((idx & (R // 2 - 1)) << 1) | ((idx >> sh) & 1)


@functools.lru_cache(maxsize=None)
def _make_sc_gather_unpack(V, O, N):
    H = O // 2
    info = plsc.get_sparse_core_info()
    NC, NS = info.num_cores, info.num_subcores
    NW = NC * NS
    n_per_w = N // NW
    CH = 256
    IR = CH // 128
    n_chunks = n_per_w // CH
    ir_per_w = n_per_w // 128
    QH = H // 16
    mesh = plsc.VectorSubcoreMesh(core_axis_name="c", subcore_axis_name="s")

    @functools.partial(
        pl.kernel,
        mesh=mesh,
        out_type=jax.ShapeDtypeStruct((N, O), jnp.float32),
        scratch_types=[
            pltpu.VMEM((ir_per_w, 128), jnp.int32),
            pltpu.VMEM((2, CH, H), jnp.int32),
            pltpu.VMEM((2, CH, O), jnp.float32),
            pltpu.SemaphoreType.DMA,
            pltpu.SemaphoreType.DMA,
        ],
        compiler_params=pltpu.CompilerParams(
            use_tc_tiling_on_sc=False, needs_layout_passes=False
        ),
    )
    def gather(packed_hbm, idx_hbm, out_hbm, idx_v, pk_v, rows_v, gsem, ssem):
        wid = lax.axis_index("s") * NC + lax.axis_index("c")
        mask_hi = jnp.full((16,), -65536, jnp.int32)
        n_pairs = n_chunks // 2

        def start_chunk(k, buf):
            for j in range(IR):
                pltpu.async_copy(
                    packed_hbm.at[idx_v.at[k * IR + j]],
                    pk_v.at[buf, pl.ds(j * 128, 128)],
                    gsem,
                )

        def drain_store(buf):
            pltpu.make_async_copy(
                out_hbm.at[pl.ds(0, CH)], rows_v.at[buf], ssem
            ).wait()

        def drain_unpack_store(k, buf):
            for j in range(IR):
                pltpu.make_async_copy(
                    packed_hbm.at[pl.ds(0, 128)],
                    pk_v.at[buf, pl.ds(j * 128, 128)],
                    gsem,
                ).wait()

            @plsc.parallel_loop(0, CH, 1, unroll=8)
            def _(r):
                for q in range(QH):
                    w = pk_v[buf, r, pl.ds(q * 16, 16)]
                    rows_v[buf, r, pl.ds(q * 16, 16)] = plsc.bitcast(
                        w << 16, jnp.float32
                    )
                    rows_v[buf, r, pl.ds(H + q * 16, 16)] = plsc.bitcast(
                        w & mask_hi, jnp.float32
                    )
            rbase = wid * n_per_w + k * CH
            pltpu.async_copy(rows_v.at[buf], out_hbm.at[pl.ds(rbase, CH)], ssem)

        def pair(kk, carry):
            k0 = kk * 2
            start_chunk(k0 + 1, 1)

            @pl.when(kk > 0)
            def _():
                drain_store(0)

            drain_unpack_store(k0, 0)

            @pl.when(kk < n_pairs - 1)
            def _():
                start_chunk(k0 + 2, 0)

            @pl.when(kk > 0)
            def _():
                drain_store(1)

            drain_unpack_store(k0 + 1, 1)
            return carry

        pltpu.sync_copy(idx_hbm.at[pl.ds(wid * ir_per_w, ir_per_w)], idx_v)
        start_chunk(0, 0)
        lax.fori_loop(0, n_pairs, pair, 0)
        drain_store(0)
        drain_store(1)

    return gather


def kernel(t_id, table, W, b):
    B, L = t_id.shape
    V, D = table.shape
    O = W.shape[0]
    N = B * L
    packed = _project_pack(table.T, W.T, b.reshape(1, O))
    pview = packed.reshape(packed.shape[0] * 2, O // 2)
    idx = _remap_idx(t_id.T.astype(jnp.int32)).reshape(N // 128, 128)
    outT = _make_sc_gather_unpack(pview.shape[0], O, N)(pview, idx)
    return outT.reshape(L, B, O).transpose(1, 0, 2)


# speedup vs baseline: 2.1413x; 1.0233x over previous
import functools

import jax
import jax.numpy as jnp
from jax import lax
from jax.experimental import pallas as pl
from jax.experimental.pallas import tpu as pltpu
from jax.experimental.pallas import tpu_sc as plsc


_RB = 32768


def _project_pack(tT, Wt, b2):
    D, V = tT.shape
    O = Wt.shape[1]
    H = O // 2
    R = _RB
    NB = pl.cdiv(V, R)

    def body(x_ref, wt_ref, b_ref, o_ref):
        xb = x_ref[...].astype(jnp.bfloat16)
        wb = wt_ref[...].astype(jnp.bfloat16)

        def pack_half(xh):
            z = (
                lax.dot_general(
                    xh, wb, (((0,), (0,)), ((), ())),
                    preferred_element_type=jnp.float32,
                )
                + b_ref[...]
            )
            u = lax.bitcast_convert_type(z, jnp.uint32)
            return lax.bitcast_convert_type(
                (u[:, :H] >> 16) | (u[:, H:] & jnp.uint32(0xFFFF0000)),
                jnp.int32,
            )

        o_ref[:, :H] = pack_half(xb[:, : R // 2])
        o_ref[:, H:] = pack_half(xb[:, R // 2 :])

    return pl.pallas_call(
        body,
        grid=(NB,),
        in_specs=[
            pl.BlockSpec((D, R), lambda i: (0, i)),
            pl.BlockSpec((D, O), lambda i: (0, 0)),
            pl.BlockSpec((1, O), lambda i: (0, 0)),
        ],
        out_specs=pl.BlockSpec((R // 2, 2 * H), lambda i: (i, 0)),
        out_shape=jax.ShapeDtypeStruct((NB * (R // 2), 2 * H), jnp.int32),
    )(tT, Wt, b2)


def _remap_idx(idx):
    R = _RB
    sh = (R // 2).bit_length() - 1
    return (idx & ~(R - 1)) | ((idx & (R // 2 - 1)) << 1) | ((idx >> sh) & 1)


@functools.lru_cache(maxsize=None)
def _make_sc_gather_unpack(V, O, N):
    H = O // 2
    info = plsc.get_sparse_core_info()
    NC, NS = info.num_cores, info.num_subcores
    NW = NC * NS
    n_per_w = N // NW
    CH = 256
    IR = CH // 128
    n_chunks = n_per_w // CH
    ir_per_w = n_per_w // 128
    QH = H // 16
    mesh = plsc.VectorSubcoreMesh(core_axis_name="c", subcore_axis_name="s")

    @functools.partial(
        pl.kernel,
        mesh=mesh,
        out_type=jax.ShapeDtypeStruct((N, O), jnp.float32),
        scratch_types=[
            pltpu.VMEM((ir_per_w, 128), jnp.int32),
            pltpu.VMEM((2, CH, H), jnp.int32),
            pltpu.VMEM((2, CH, O), jnp.float32),
            pltpu.SemaphoreType.DMA,
            pltpu.SemaphoreType.DMA,
        ],
        compiler_params=pltpu.CompilerParams(
            use_tc_tiling_on_sc=False, needs_layout_passes=False
        ),
    )
    def gather(packed_hbm, idx_hbm, out_hbm, idx_v, pk_v, rows_v, gsem, ssem):
        wid = lax.axis_index("s") * NC + lax.axis_index("c")
        mask_hi = jnp.full((16,), -65536, jnp.int32)
        n_pairs = n_chunks // 2

        def start_chunk(k, buf):
            for j in range(IR):
                pltpu.async_copy(
                    packed_hbm.at[idx_v.at[k * IR + j]],
                    pk_v.at[buf, pl.ds(j * 128, 128)],
                    gsem,
                )

        def drain_store(buf):
            pltpu.make_async_copy(
                out_hbm.at[pl.ds(0, CH)], rows_v.at[buf], ssem
            ).wait()

        def drain_unpack_store(k, buf):
            for j in range(IR):
                pltpu.make_async_copy(
                    packed_hbm.at[pl.ds(0, 128)],
                    pk_v.at[buf, pl.ds(j * 128, 128)],
                    gsem,
                ).wait()

            @plsc.parallel_loop(0, CH, 1, unroll=8)
            def _(r):
                for q in range(QH):
                    w = pk_v[buf, r, pl.ds(q * 16, 16)]
                    rows_v[buf, r, pl.ds(q * 16, 16)] = plsc.bitcast(
                        w << 16, jnp.float32
                    )
                    rows_v[buf, r, pl.ds(H + q * 16, 16)] = plsc.bitcast(
                        w & mask_hi, jnp.float32
                    )
            rbase = wid * n_per_w + k * CH
            pltpu.async_copy(rows_v.at[buf], out_hbm.at[pl.ds(rbase, CH)], ssem)

        def pair(kk, carry):
            k0 = kk * 2
            start_chunk(k0 + 1, 1)

            @pl.when(kk > 0)
            def _():
                drain_store(0)

            drain_unpack_store(k0, 0)

            @pl.when(kk < n_pairs - 1)
            def _():
                start_chunk(k0 + 2, 0)

            @pl.when(kk > 0)
            def _():
                drain_store(1)

            drain_unpack_store(k0 + 1, 1)
            return carry

        pltpu.sync_copy(idx_hbm.at[pl.ds(wid * ir_per_w, ir_per_w)], idx_v)
        start_chunk(0, 0)
        lax.fori_loop(0, n_pairs, pair, 0)
        drain_store(0)
        drain_store(1)

    return gather


def kernel(t_id, table, W, b):
    B, L = t_id.shape
    V, D = table.shape
    O = W.shape[0]
    N = B * L
    packed = _project_pack(table.T, W.T, b.reshape(1, O))
    pview = packed.reshape(packed.shape[0] * 2, O // 2)
    idx = _remap_idx(t_id.T.astype(jnp.int32)).reshape(N // 128, 128)
    outT = _make_sc_gather_unpack(pview.shape[0], O, N)(pview, idx)
    return outT.reshape(L, B, O).transpose(1, 0, 2)
